# baseline (speedup 1.0000x reference)
import functools

import jax
import jax.numpy as jnp
from jax import lax
from jax.experimental import pallas as pl
from jax.experimental.pallas import tpu as pltpu
from jax.experimental.pallas import tpu_sc as plsc

MARGIN = 1.0
N = 4096
NPROP = 128
L = 16
NC, NS = 2, 16
NW = NC * NS
PPW = NPROP // NW
NCHUNK = N // L
BUF = N + L
NEG_PAD = -1.0e30


def _sc_body(scores_hbm, labels_hbm, props_hbm, out_hbm,
             scores_v, labels_v, props_v, part_v, *bufs):
    pos_bufs = bufs[:PPW]
    neg_bufs = bufs[PPW:]
    wid = lax.axis_index("c") * NS + lax.axis_index("s")
    base_prop = wid * PPW

    pltpu.sync_copy(scores_hbm, scores_v)
    pltpu.sync_copy(labels_hbm, labels_v)
    pltpu.sync_copy(props_hbm, props_v)

    def chunk_body(k, offs):
        sl = pl.ds(k * L, L)
        p = props_v[sl]
        mine = (p >> 2) == wid
        any_mine = plsc.all_reduce_population_count(mine)[0] > 0

        def taken(offs):
            s = scores_v[sl]
            is_pos = labels_v[sl] == 1
            new_offs = list(offs)
            for t in range(PPW):
                m_same = p == (base_prop + t)
                m_pos = m_same & is_pos
                m_neg = m_same & (~is_pos)
                npos_h = plsc.all_reduce_population_count(m_pos)[0]
                nneg_h = plsc.all_reduce_population_count(m_neg)[0]

                def do_t(op, on, m_pos=m_pos, m_neg=m_neg, s=s):
                    cum_pos = plsc.cumsum(m_pos.astype(jnp.int32))
                    cum_neg = plsc.cumsum(m_neg.astype(jnp.int32))
                    idx_pos = op + jnp.maximum(cum_pos - 1, 0)
                    idx_neg = on + jnp.maximum(cum_neg - 1, 0)
                    plsc.store_scatter(pos_bufs[t], [idx_pos], s, mask=m_pos)
                    plsc.store_scatter(neg_bufs[t], [idx_neg], s, mask=m_neg)
                    return op, on

                lax.cond(npos_h + nneg_h > 0, do_t,
                         lambda op, on: (op, on), offs[t], offs[PPW + t])
                new_offs[t] = offs[t] + npos_h
                new_offs[PPW + t] = offs[PPW + t] + nneg_h
            return tuple(new_offs)

        return lax.cond(any_mine, taken, lambda o: o, offs)

    zero = jnp.int32(0)
    counts = lax.fori_loop(0, NCHUNK, chunk_body, (zero,) * (2 * PPW))

    pad_vec = jnp.full((L,), NEG_PAD, jnp.float32)
    acc = jnp.zeros((L,), jnp.float32)
    pairs = zero
    for t in range(PPW):
        npos, nneg = counts[t], counts[PPW + t]
        neg_bufs[t][pl.ds(nneg, L)] = pad_vec
        pairs = pairs + npos * nneg
        nch = (nneg + (L - 1)) // L

        def pos_body(i, a, t=t, nch=nch):
            coef = MARGIN - pos_bufs[t][pl.ds(i, L)][0]

            def neg_body(c, aa, t=t, coef=coef):
                nv = neg_bufs[t][pl.ds(c * L, L)]
                return aa + jnp.maximum(coef + nv, 0.0)

            return lax.fori_loop(0, nch, neg_body, a)

        acc = lax.fori_loop(0, npos, pos_body, acc)

    loss = jnp.sum(acc)
    lane = lax.broadcasted_iota(jnp.int32, (L,), 0)
    part = jnp.where(lane == 0, loss,
                     jnp.where(lane == 1, pairs.astype(jnp.float32), 0.0))
    part_v[...] = part
    pltpu.sync_copy(part_v, out_hbm.at[wid])


@jax.jit
def _pairwise_loss_sc(scores, labels, props):
    mesh = plsc.VectorSubcoreMesh(core_axis_name="c", subcore_axis_name="s")
    scratch = [
        pltpu.VMEM((N,), jnp.float32),
        pltpu.VMEM((N,), jnp.int32),
        pltpu.VMEM((N,), jnp.int32),
        pltpu.VMEM((L,), jnp.float32),
    ] + [pltpu.VMEM((BUF,), jnp.float32) for _ in range(2 * PPW)]
    parts = pl.kernel(
        _sc_body,
        out_type=jax.ShapeDtypeStruct((NW, L), jnp.float32),
        mesh=mesh,
        scratch_types=scratch,
        compiler_params=pltpu.CompilerParams(needs_layout_passes=False),
    )(scores, labels, props)
    loss = parts[:, 0].sum()
    pairs = parts[:, 1].sum()
    return jnp.where(pairs == 0.0, 0.0, loss / jnp.maximum(pairs, 1.0))


def kernel(scores, labels, property_ids):
    scores = scores.reshape(-1).astype(jnp.float32)
    labels = labels.reshape(-1).astype(jnp.int32)
    props = property_ids.reshape(-1).astype(jnp.int32)
    return _pairwise_loss_sc(scores, labels, props)


# speedup vs baseline: 1.0267x; 1.0267x over previous
import functools

import jax
import jax.numpy as jnp
from jax import lax
from jax.experimental import pallas as pl
from jax.experimental.pallas import tpu as pltpu
from jax.experimental.pallas import tpu_sc as plsc

MARGIN = 1.0
N = 4096
NPROP = 128
L = 16
NC, NS = 2, 16
NW = NC * NS
PPW = NPROP // NW
NCHUNK = N // L
BUF = N + L
NEG_PAD = -1.0e30


def _sc_body(scores_hbm, labels_hbm, props_hbm, out_hbm,
             scores_v, labels_v, props_v, part_v, *bufs):
    pos_bufs = bufs[:PPW]
    neg_bufs = bufs[PPW:]
    wid = lax.axis_index("c") * NS + lax.axis_index("s")
    base_prop = wid * PPW

    pltpu.sync_copy(scores_hbm, scores_v)
    pltpu.sync_copy(labels_hbm, labels_v)
    pltpu.sync_copy(props_hbm, props_v)

    def chunk_body(k, offs):
        sl = pl.ds(k * L, L)
        p = props_v[sl]
        mine = (p >> 2) == wid
        any_mine = plsc.all_reduce_population_count(mine)[0] > 0

        def taken(offs):
            s = scores_v[sl]
            is_pos = labels_v[sl] == 1
            new_offs = [None] * (2 * PPW)
            for t in range(PPW):
                m_same = p == (base_prop + t)
                m_pos = m_same & is_pos
                m_neg = m_same & (~is_pos)
                cum_pos = plsc.cumsum(m_pos.astype(jnp.int32))
                cum_neg = plsc.cumsum(m_neg.astype(jnp.int32))
                idx_pos = offs[t] + jnp.maximum(cum_pos - 1, 0)
                idx_neg = offs[PPW + t] + jnp.maximum(cum_neg - 1, 0)
                plsc.store_scatter(pos_bufs[t], [idx_pos], s, mask=m_pos)
                plsc.store_scatter(neg_bufs[t], [idx_neg], s, mask=m_neg)
                new_offs[t] = offs[t] + cum_pos[L - 1]
                new_offs[PPW + t] = offs[PPW + t] + cum_neg[L - 1]
            return tuple(new_offs)

        return lax.cond(any_mine, taken, lambda o: o, offs)

    zero = jnp.int32(0)
    counts = lax.fori_loop(0, NCHUNK, chunk_body, (zero,) * (2 * PPW))

    pad_vec = jnp.full((L,), NEG_PAD, jnp.float32)
    acc = jnp.zeros((L,), jnp.float32)
    pairs = zero
    for t in range(PPW):
        npos, nneg = counts[t], counts[PPW + t]
        neg_bufs[t][pl.ds(nneg, L)] = pad_vec
        pairs = pairs + npos * nneg
        nch = (nneg + (L - 1)) // L

        def pos_body(i, a, t=t, nch=nch):
            coef = MARGIN - pos_bufs[t][pl.ds(i, L)][0]

            def neg_body(c, aa, t=t, coef=coef):
                nv = neg_bufs[t][pl.ds(c * L, L)]
                return aa + jnp.maximum(coef + nv, 0.0)

            return lax.fori_loop(0, nch, neg_body, a)

        acc = lax.fori_loop(0, npos, pos_body, acc)

    loss = jnp.sum(acc)
    lane = lax.broadcasted_iota(jnp.int32, (L,), 0)
    part = jnp.where(lane == 0, loss,
                     jnp.where(lane == 1, pairs.astype(jnp.float32), 0.0))
    part_v[...] = part
    pltpu.sync_copy(part_v, out_hbm.at[wid])


@jax.jit
def _pairwise_loss_sc(scores, labels, props):
    mesh = plsc.VectorSubcoreMesh(core_axis_name="c", subcore_axis_name="s")
    scratch = [
        pltpu.VMEM((N,), jnp.float32),
        pltpu.VMEM((N,), jnp.int32),
        pltpu.VMEM((N,), jnp.int32),
        pltpu.VMEM((L,), jnp.float32),
    ] + [pltpu.VMEM((BUF,), jnp.float32) for _ in range(2 * PPW)]
    parts = pl.kernel(
        _sc_body,
        out_type=jax.ShapeDtypeStruct((NW, L), jnp.float32),
        mesh=mesh,
        scratch_types=scratch,
        compiler_params=pltpu.CompilerParams(needs_layout_passes=False),
    )(scores, labels, props)
    loss = parts[:, 0].sum()
    pairs = parts[:, 1].sum()
    return jnp.where(pairs == 0.0, 0.0, loss / jnp.maximum(pairs, 1.0))


def kernel(scores, labels, property_ids):
    scores = scores.reshape(-1).astype(jnp.float32)
    labels = labels.reshape(-1).astype(jnp.int32)
    props = property_ids.reshape(-1).astype(jnp.int32)
    return _pairwise_loss_sc(scores, labels, props)


# speedup vs baseline: 1.1497x; 1.1198x over previous
import functools

import jax
import jax.numpy as jnp
from jax import lax
from jax.experimental import pallas as pl
from jax.experimental.pallas import tpu as pltpu
from jax.experimental.pallas import tpu_sc as plsc

MARGIN = 1.0
N = 4096
NPROP = 128
L = 16
NC, NS = 2, 16
NW = NC * NS
PPW = NPROP // NW
NB = 2 * PPW
NCHUNK = N // L
CAP = N + L
NEG_PAD = -1.0e30


def _sc_body(scores_hbm, labels_hbm, props_hbm, out_hbm,
             scores_v, labels_v, props_v, part_v, off_v, big_v):
    wid = lax.axis_index("c") * NS + lax.axis_index("s")

    pltpu.sync_copy(scores_hbm, scores_v)
    pltpu.sync_copy(labels_hbm, labels_v)
    pltpu.sync_copy(props_hbm, props_v)

    off_v[pl.ds(0, L)] = jnp.zeros((L,), jnp.int32)

    def chunk_body(k, carry):
        sl = pl.ds(k * L, L)
        p = props_v[sl]
        mine = (p >> 2) == wid
        key = (p << 1) | labels_v[sl]
        t_idx = key & (NB - 1)
        rank, last = plsc.scan_count(key, mask=mine)
        base = plsc.load_gather(off_v, [t_idx])
        addr = t_idx * CAP + base + (rank - 1)
        plsc.store_scatter(big_v, [addr], scores_v[sl], mask=mine)
        plsc.addupdate_scatter(off_v, [t_idx], rank, mask=last & mine)
        return carry

    lax.fori_loop(0, NCHUNK, chunk_body, jnp.int32(0))
    cnt = off_v[pl.ds(0, L)]

    pad_vec = jnp.full((L,), NEG_PAD, jnp.float32)
    acc = jnp.zeros((L,), jnp.float32)
    pairs = jnp.int32(0)
    for t in range(PPW):
        nneg, npos = cnt[2 * t], cnt[2 * t + 1]
        negbase, posbase = (2 * t) * CAP, (2 * t + 1) * CAP
        big_v[pl.ds(negbase + nneg, L)] = pad_vec
        pairs = pairs + npos * nneg
        nch = (nneg + (L - 1)) // L

        def pos_body(i, a, posbase=posbase, negbase=negbase, nch=nch):
            coef = MARGIN - big_v[pl.ds(posbase + i, L)][0]

            def neg_body(c, aa, negbase=negbase, coef=coef):
                nv = big_v[pl.ds(negbase + c * L, L)]
                return aa + jnp.maximum(coef + nv, 0.0)

            return lax.fori_loop(0, nch, neg_body, a)

        acc = lax.fori_loop(0, npos, pos_body, acc)

    loss = jnp.sum(acc)
    lane = lax.broadcasted_iota(jnp.int32, (L,), 0)
    part = jnp.where(lane == 0, loss,
                     jnp.where(lane == 1, pairs.astype(jnp.float32), 0.0))
    part_v[...] = part
    pltpu.sync_copy(part_v, out_hbm.at[wid])


@jax.jit
def _pairwise_loss_sc(scores, labels, props):
    mesh = plsc.VectorSubcoreMesh(core_axis_name="c", subcore_axis_name="s")
    scratch = [
        pltpu.VMEM((N,), jnp.float32),
        pltpu.VMEM((N,), jnp.int32),
        pltpu.VMEM((N,), jnp.int32),
        pltpu.VMEM((L,), jnp.float32),
        pltpu.VMEM((L,), jnp.int32),
        pltpu.VMEM((NB * CAP,), jnp.float32),
    ]
    parts = pl.kernel(
        _sc_body,
        out_type=jax.ShapeDtypeStruct((NW, L), jnp.float32),
        mesh=mesh,
        scratch_types=scratch,
        compiler_params=pltpu.CompilerParams(needs_layout_passes=False),
    )(scores, labels, props)
    loss = parts[:, 0].sum()
    pairs = parts[:, 1].sum()
    return jnp.where(pairs == 0.0, 0.0, loss / jnp.maximum(pairs, 1.0))


def kernel(scores, labels, property_ids):
    scores = scores.reshape(-1).astype(jnp.float32)
    labels = labels.reshape(-1).astype(jnp.int32)
    props = property_ids.reshape(-1).astype(jnp.int32)
    return _pairwise_loss_sc(scores, labels, props)


# speedup vs baseline: 1.1535x; 1.0033x over previous
import functools

import jax
import jax.numpy as jnp
from jax import lax
from jax.experimental import pallas as pl
from jax.experimental.pallas import tpu as pltpu
from jax.experimental.pallas import tpu_sc as plsc

MARGIN = 1.0
N = 4096
NPROP = 128
L = 16
NC, NS = 2, 16
NW = NC * NS
PPW = NPROP // NW
NB = 2 * PPW
NCHUNK = N // L
CAP = N + L
NEG_PAD = -1.0e30


def _sc_body(scores_hbm, labels_hbm, props_hbm, out_hbm,
             scores_v, labels_v, props_v, part_v, off_v, big_v):
    wid = lax.axis_index("c") * NS + lax.axis_index("s")

    pltpu.sync_copy(scores_hbm, scores_v)
    pltpu.sync_copy(labels_hbm, labels_v)
    pltpu.sync_copy(props_hbm, props_v)

    off_v[pl.ds(0, L)] = jnp.zeros((L,), jnp.int32)
    off_v[pl.ds(L, L)] = jnp.zeros((L,), jnp.int32)

    def chunk_body(k, carry):
        sl = pl.ds(k * L, L)
        p = props_v[sl]
        mine = (p >> 2) == wid
        key = (p << 1) | labels_v[sl]
        t_idx = key & (NB - 1)
        rank, last = plsc.scan_count(key, mask=mine)
        base = plsc.load_gather(off_v, [t_idx])
        addr = t_idx * CAP + base + (rank - 1)
        plsc.store_scatter(big_v, [addr], scores_v[sl], mask=mine)
        plsc.addupdate_scatter(off_v, [t_idx], rank, mask=last & mine)
        return carry

    lax.fori_loop(0, NCHUNK, chunk_body, jnp.int32(0))

    pad_vec = jnp.full((L,), NEG_PAD, jnp.float32)

    def t_body(t, carry):
        acc, pairs = carry
        offs = off_v[pl.ds(2 * t, L)]
        nneg, npos = offs[0], offs[1]
        negbase = (2 * t) * CAP
        posbase = negbase + CAP
        big_v[pl.ds(negbase + nneg, L)] = pad_vec
        pairs = pairs + npos * nneg
        nch = (nneg + (L - 1)) // L

        def pos_body(i, a):
            coef = MARGIN - big_v[pl.ds(posbase + i, L)][0]

            def neg_body(c, aa):
                nv = big_v[pl.ds(negbase + c * L, L)]
                return aa + jnp.maximum(coef + nv, 0.0)

            return lax.fori_loop(0, nch, neg_body, a)

        return lax.fori_loop(0, npos, pos_body, acc), pairs

    acc, pairs = lax.fori_loop(
        0, PPW, t_body, (jnp.zeros((L,), jnp.float32), jnp.int32(0)))

    loss = jnp.sum(acc)
    lane = lax.broadcasted_iota(jnp.int32, (L,), 0)
    part = jnp.where(lane == 0, loss,
                     jnp.where(lane == 1, pairs.astype(jnp.float32), 0.0))
    part_v[...] = part
    pltpu.sync_copy(part_v, out_hbm.at[wid])


@jax.jit
def _pairwise_loss_sc(scores, labels, props):
    mesh = plsc.VectorSubcoreMesh(core_axis_name="c", subcore_axis_name="s")
    scratch = [
        pltpu.VMEM((N,), jnp.float32),
        pltpu.VMEM((N,), jnp.int32),
        pltpu.VMEM((N,), jnp.int32),
        pltpu.VMEM((L,), jnp.float32),
        pltpu.VMEM((2 * L,), jnp.int32),
        pltpu.VMEM((NB * CAP,), jnp.float32),
    ]
    parts = pl.kernel(
        _sc_body,
        out_type=jax.ShapeDtypeStruct((NW, L), jnp.float32),
        mesh=mesh,
        scratch_types=scratch,
        compiler_params=pltpu.CompilerParams(needs_layout_passes=False),
    )(scores, labels, props)
    loss = parts[:, 0].sum()
    pairs = parts[:, 1].sum()
    return jnp.where(pairs == 0.0, 0.0, loss / jnp.maximum(pairs, 1.0))


def kernel(scores, labels, property_ids):
    scores = scores.reshape(-1).astype(jnp.float32)
    labels = labels.reshape(-1).astype(jnp.int32)
    props = property_ids.reshape(-1).astype(jnp.int32)
    return _pairwise_loss_sc(scores, labels, props)


# speedup vs baseline: 1.1574x; 1.0034x over previous
import functools

import jax
import jax.numpy as jnp
from jax import lax
from jax.experimental import pallas as pl
from jax.experimental.pallas import tpu as pltpu
from jax.experimental.pallas import tpu_sc as plsc

MARGIN = 1.0
N = 4096
NPROP = 128
L = 16
NC, NS = 2, 16
NW = NC * NS
PPW = NPROP // NW
NB = 2 * PPW
NCHUNK = N // L
CAP = N + L
NEG_PAD = -1.0e30


def _sc_body(scores_hbm, labels_hbm, props_hbm, out_hbm,
             scores_v, labels_v, props_v, part_v, off_v, big_v):
    wid = lax.axis_index("c") * NS + lax.axis_index("s")

    pltpu.sync_copy(scores_hbm, scores_v)
    pltpu.sync_copy(labels_hbm, labels_v)
    pltpu.sync_copy(props_hbm, props_v)

    off_v[pl.ds(0, L)] = jnp.zeros((L,), jnp.int32)
    off_v[pl.ds(L, L)] = jnp.zeros((L,), jnp.int32)

    lane = lax.broadcasted_iota(jnp.int32, (L,), 0)

    def chunk_body(k, offs_vec):
        sl = pl.ds(k * L, L)
        p = props_v[sl]
        mine = (p >> 2) == wid
        key = (p << 1) | labels_v[sl]
        t_idx = key & (NB - 1)
        rank, _ = plsc.scan_count(key, mask=mine)
        base = lax.gather(
            offs_vec, t_idx[:, None],
            lax.GatherDimensionNumbers(
                offset_dims=(), collapsed_slice_dims=(0,),
                start_index_map=(0,)),
            slice_sizes=(1,),
            mode=lax.GatherScatterMode.PROMISE_IN_BOUNDS)
        addr = t_idx * CAP + base + (rank - 1)
        plsc.store_scatter(big_v, [addr], scores_v[sl], mask=mine)
        delta = jnp.zeros((L,), jnp.int32)
        for b in range(NB):
            cb = plsc.all_reduce_population_count(mine & (t_idx == b))
            delta = jnp.where(lane == b, delta + cb, delta)
        return offs_vec + delta

    offs_vec = lax.fori_loop(0, NCHUNK, chunk_body,
                             jnp.zeros((L,), jnp.int32))
    off_v[pl.ds(0, L)] = offs_vec

    pad_vec = jnp.full((L,), NEG_PAD, jnp.float32)

    def t_body(t, carry):
        acc, pairs = carry
        offs = off_v[pl.ds(2 * t, L)]
        nneg, npos = offs[0], offs[1]
        negbase = (2 * t) * CAP
        posbase = negbase + CAP
        big_v[pl.ds(negbase + nneg, L)] = pad_vec
        pairs = pairs + npos * nneg
        nch = (nneg + (L - 1)) // L

        def pos_body(i, a):
            coef = MARGIN - big_v[pl.ds(posbase + i, L)][0]

            def neg_body(c, aa):
                nv = big_v[pl.ds(negbase + c * L, L)]
                return aa + jnp.maximum(coef + nv, 0.0)

            return lax.fori_loop(0, nch, neg_body, a)

        return lax.fori_loop(0, npos, pos_body, acc), pairs

    acc, pairs = lax.fori_loop(
        0, PPW, t_body, (jnp.zeros((L,), jnp.float32), jnp.int32(0)))

    loss = jnp.sum(acc)
    lane = lax.broadcasted_iota(jnp.int32, (L,), 0)
    part = jnp.where(lane == 0, loss,
                     jnp.where(lane == 1, pairs.astype(jnp.float32), 0.0))
    part_v[...] = part
    pltpu.sync_copy(part_v, out_hbm.at[wid])


@jax.jit
def _pairwise_loss_sc(scores, labels, props):
    mesh = plsc.VectorSubcoreMesh(core_axis_name="c", subcore_axis_name="s")
    scratch = [
        pltpu.VMEM((N,), jnp.float32),
        pltpu.VMEM((N,), jnp.int32),
        pltpu.VMEM((N,), jnp.int32),
        pltpu.VMEM((L,), jnp.float32),
        pltpu.VMEM((2 * L,), jnp.int32),
        pltpu.VMEM((NB * CAP,), jnp.float32),
    ]
    parts = pl.kernel(
        _sc_body,
        out_type=jax.ShapeDtypeStruct((NW, L), jnp.float32),
        mesh=mesh,
        scratch_types=scratch,
        compiler_params=pltpu.CompilerParams(needs_layout_passes=False),
    )(scores, labels, props)
    loss = parts[:, 0].sum()
    pairs = parts[:, 1].sum()
    return jnp.where(pairs == 0.0, 0.0, loss / jnp.maximum(pairs, 1.0))


def kernel(scores, labels, property_ids):
    scores = scores.reshape(-1).astype(jnp.float32)
    labels = labels.reshape(-1).astype(jnp.int32)
    props = property_ids.reshape(-1).astype(jnp.int32)
    return _pairwise_loss_sc(scores, labels, props)


# speedup vs baseline: 1.1889x; 1.0272x over previous
import functools

import jax
import jax.numpy as jnp
from jax import lax
from jax.experimental import pallas as pl
from jax.experimental.pallas import tpu as pltpu
from jax.experimental.pallas import tpu_sc as plsc

MARGIN = 1.0
N = 4096
NPROP = 128
L = 16
NC, NS = 2, 16
NW = NC * NS
PPW = NPROP // NW
NB = 2 * PPW
NCHUNK = N // L
CAP = N + L
NEG_PAD = -1.0e30


def _sc_body(scores_hbm, labels_hbm, props_hbm, out_hbm,
             scores_v, labels_v, props_v, part_v, off_v, big_v):
    wid = lax.axis_index("c") * NS + lax.axis_index("s")

    pltpu.sync_copy(scores_hbm, scores_v)
    pltpu.sync_copy(labels_hbm, labels_v)
    pltpu.sync_copy(props_hbm, props_v)

    off_v[pl.ds(0, L)] = jnp.zeros((L,), jnp.int32)
    off_v[pl.ds(L, L)] = jnp.zeros((L,), jnp.int32)

    lane = lax.broadcasted_iota(jnp.int32, (L,), 0)

    @plsc.parallel_loop(0, NCHUNK, carry=jnp.zeros((L,), jnp.int32),
                        unroll=2)
    def offs_vec(k, offs_vec):
        sl = pl.ds(k * L, L)
        p = props_v[sl]
        mine = (p >> 2) == wid
        key = (p << 1) | labels_v[sl]
        t_idx = key & (NB - 1)
        rank, _ = plsc.scan_count(key, mask=mine)
        base = lax.gather(
            offs_vec, t_idx[:, None],
            lax.GatherDimensionNumbers(
                offset_dims=(), collapsed_slice_dims=(0,),
                start_index_map=(0,)),
            slice_sizes=(1,),
            mode=lax.GatherScatterMode.PROMISE_IN_BOUNDS)
        addr = t_idx * CAP + base + (rank - 1)
        plsc.store_scatter(big_v, [addr], scores_v[sl], mask=mine)
        delta = jnp.zeros((L,), jnp.int32)
        for b in range(NB):
            cb = plsc.all_reduce_population_count(mine & (t_idx == b))
            delta = jnp.where(lane == b, delta + cb, delta)
        return offs_vec + delta

    off_v[pl.ds(0, L)] = offs_vec

    pad_vec = jnp.full((L,), NEG_PAD, jnp.float32)

    def t_body(t, carry):
        acc, pairs = carry
        offs = off_v[pl.ds(2 * t, L)]
        nneg, npos = offs[0], offs[1]
        negbase = (2 * t) * CAP
        posbase = negbase + CAP
        big_v[pl.ds(negbase + nneg, L)] = pad_vec
        pairs = pairs + npos * nneg
        nch = (nneg + (L - 1)) // L

        def pos_body(i, a):
            coef = MARGIN - big_v[pl.ds(posbase + i, L)][0]

            def neg_body(c, aa):
                nv = big_v[pl.ds(negbase + c * L, L)]
                return aa + jnp.maximum(coef + nv, 0.0)

            return lax.fori_loop(0, nch, neg_body, a)

        return lax.fori_loop(0, npos, pos_body, acc), pairs

    acc, pairs = lax.fori_loop(
        0, PPW, t_body, (jnp.zeros((L,), jnp.float32), jnp.int32(0)))

    loss = jnp.sum(acc)
    lane = lax.broadcasted_iota(jnp.int32, (L,), 0)
    part = jnp.where(lane == 0, loss,
                     jnp.where(lane == 1, pairs.astype(jnp.float32), 0.0))
    part_v[...] = part
    pltpu.sync_copy(part_v, out_hbm.at[wid])


@jax.jit
def _pairwise_loss_sc(scores, labels, props):
    mesh = plsc.VectorSubcoreMesh(core_axis_name="c", subcore_axis_name="s")
    scratch = [
        pltpu.VMEM((N,), jnp.float32),
        pltpu.VMEM((N,), jnp.int32),
        pltpu.VMEM((N,), jnp.int32),
        pltpu.VMEM((L,), jnp.float32),
        pltpu.VMEM((2 * L,), jnp.int32),
        pltpu.VMEM((NB * CAP,), jnp.float32),
    ]
    parts = pl.kernel(
        _sc_body,
        out_type=jax.ShapeDtypeStruct((NW, L), jnp.float32),
        mesh=mesh,
        scratch_types=scratch,
        compiler_params=pltpu.CompilerParams(needs_layout_passes=False),
    )(scores, labels, props)
    loss = parts[:, 0].sum()
    pairs = parts[:, 1].sum()
    return jnp.where(pairs == 0.0, 0.0, loss / jnp.maximum(pairs, 1.0))


def kernel(scores, labels, property_ids):
    scores = scores.reshape(-1).astype(jnp.float32)
    labels = labels.reshape(-1).astype(jnp.int32)
    props = property_ids.reshape(-1).astype(jnp.int32)
    return _pairwise_loss_sc(scores, labels, props)
